# baseline (speedup 1.0000x reference)
import functools

import jax
import jax.numpy as jnp
from jax import lax
from jax.experimental import pallas as pl
from jax.experimental.pallas import tpu as pltpu
from jax.experimental.pallas import tpu_sc as plsc

VSIZE = 1000000
ESIZE = 16
PK = 2 * ESIZE
SENTLEN = 50
RAD = 5
NSAMPL = 5
BATCH = 4096

NHALF = 2
BATCH_H = BATCH // NHALF

NC, NS = 2, 16
NW = NC * NS
CHUNK = 128
NCHUNK = 64
WINDOW = 16

ROWS_W = BATCH_H * SENTLEN // NW
NCH = ROWS_W // CHUNK
NROWS_NEG_W = BATCH_H * NSAMPL // NW
NCH_NEG = NROWS_NEG_W // NCHUNK


def _sc_gather_body(tab_hbm, sent_hbm, negw_hbm, cc_out, neg_out,
                    idx_v, nidx_v, rows_v, sem):
    wid = lax.axis_index("s") * NC + lax.axis_index("c")
    pltpu.sync_copy(sent_hbm.at[wid], idx_v)
    pltpu.sync_copy(negw_hbm.at[wid], nidx_v)

    def gather_to(out_hbm, idx_ref, nch, chunk):
        def body(j, carry):
            pltpu.async_copy(tab_hbm.at[idx_ref.at[j]],
                             rows_v.at[pl.ds(j * chunk, chunk)], sem)

            @pl.when(j >= WINDOW)
            def _():
                pltpu.make_async_copy(tab_hbm.at[pl.ds(0, chunk)],
                                      rows_v.at[pl.ds(0, chunk)], sem).wait()

            return carry

        lax.fori_loop(0, nch, body, 0)
        tail = min(nch, WINDOW) * chunk
        pltpu.make_async_copy(tab_hbm.at[pl.ds(0, tail)],
                              rows_v.at[pl.ds(0, tail)], sem).wait()
        pltpu.sync_copy(rows_v.at[pl.ds(0, nch * chunk)], out_hbm.at[wid])

    gather_to(cc_out, idx_v, NCH, CHUNK)
    gather_to(neg_out, nidx_v, NCH_NEG, NCHUNK)


@functools.cache
def _make_sc_gather():
    return pl.kernel(
        _sc_gather_body,
        out_type=[
            jax.ShapeDtypeStruct((NW, ROWS_W, PK), jnp.bfloat16),
            jax.ShapeDtypeStruct((NW, NROWS_NEG_W, PK), jnp.bfloat16),
        ],
        mesh=plsc.VectorSubcoreMesh(core_axis_name="c", subcore_axis_name="s",
                                    num_cores=NC, num_subcores=NS),
        scratch_types=[
            pltpu.VMEM((NCH, CHUNK), jnp.int32),
            pltpu.VMEM((NCH_NEG, NCHUNK), jnp.int32),
            pltpu.VMEM((ROWS_W, PK), jnp.bfloat16),
            pltpu.SemaphoreType.DMA,
        ],
        compiler_params=pltpu.CompilerParams(use_tc_tiling_on_sc=False),
    )


BB = 512
LP = SENTLEN * PK
NP = NSAMPL * PK


def _softplus(x):
    return jnp.log1p(jnp.exp(-jnp.abs(x))) + jnp.maximum(x, 0.0)


def _tc_loss_body(cc_ref, neg_ref, out_ref):
    i = pl.program_id(0)
    x = cc_ref[...].astype(jnp.float32)
    y = neg_ref[...].astype(jnp.float32)
    ki = lax.broadcasted_iota(jnp.int32, (LP, SENTLEN), 0)
    ji = lax.broadcasted_iota(jnp.int32, (LP, SENTLEN), 1)
    Scen = ((ki // PK == ji) & (ki % PK < ESIZE)).astype(jnp.float32)
    Scon = ((ki // PK == ji) & (ki % PK >= ESIZE)).astype(jnp.float32)
    pos = jnp.zeros((), jnp.float32)
    for d in range(1, RAD + 1):
        w1 = LP - PK * d - ESIZE
        p1 = x[:, :w1] * x[:, PK * d + ESIZE:]
        s1 = jnp.dot(p1, Scen[:w1, :SENTLEN - d],
                     preferred_element_type=jnp.float32)
        w2 = LP - PK * d + ESIZE
        p2 = x[:, :w2] * x[:, PK * d - ESIZE:]
        s2 = jnp.dot(p2, Scon[:w2, :SENTLEN - d],
                     preferred_element_type=jnp.float32)
        pos += jnp.sum(_softplus(-s1)) + jnp.sum(_softplus(-s2))
    negsum = jnp.zeros((), jnp.float32)
    kn = lax.broadcasted_iota(jnp.int32, (NP, LP), 0)
    mn = lax.broadcasted_iota(jnp.int32, (NP, LP), 1)
    for n in range(NSAMPL):
        Tn = ((kn == PK * n + ESIZE + mn % PK) & (mn % PK < ESIZE)
              ).astype(jnp.float32)
        nb = jnp.dot(y, Tn, preferred_element_type=jnp.float32)
        s = jnp.dot(x * nb, Scen, preferred_element_type=jnp.float32)
        negsum += jnp.sum(_softplus(s))
    val = (pos / (BATCH * SENTLEN * SENTLEN)
           + negsum / (BATCH * SENTLEN * NSAMPL))

    @pl.when(i == 0)
    def _():
        out_ref[...] = jnp.zeros((1, 1), jnp.float32)

    out_ref[...] = out_ref[...] + val


_tc_loss = pl.pallas_call(
    _tc_loss_body,
    grid=(BATCH_H // BB,),
    in_specs=[
        pl.BlockSpec((BB, LP), lambda i: (i, 0)),
        pl.BlockSpec((BB, NP), lambda i: (i, 0)),
    ],
    out_specs=pl.BlockSpec((1, 1), lambda i: (0, 0)),
    out_shape=jax.ShapeDtypeStruct((1, 1), jnp.float32),
)


def kernel(sent, cenb_w, cemb_w, negwords):
    a4 = cenb_w.astype(jnp.bfloat16).reshape(VSIZE // 4, 4, ESIZE)
    b4 = cemb_w.astype(jnp.bfloat16).reshape(VSIZE // 4, 4, ESIZE)
    t128 = jnp.concatenate([a4, b4], axis=2).reshape(VSIZE // 4, 128)
    t128 = jax.lax.optimization_barrier(t128)
    tab = t128.reshape(VSIZE, PK)
    sent_r = sent.astype(jnp.int32).reshape(NHALF, NW, NCH, CHUNK)
    negw_r = negwords.astype(jnp.int32).reshape(NHALF, NW, NCH_NEG, NCHUNK)
    sc = _make_sc_gather()
    total = jnp.zeros((), jnp.float32)
    for h in range(NHALF):
        cc_g, neg_g = sc(tab, sent_r[h], negw_r[h])
        cc = cc_g.reshape(BATCH_H, LP)
        neg = neg_g.reshape(BATCH_H, NP)
        total = total + _tc_loss(cc, neg)[0, 0]
    return total


# speedup vs baseline: 1.3506x; 1.3506x over previous
import functools

import jax
import jax.numpy as jnp
from jax import lax
from jax.experimental import pallas as pl
from jax.experimental.pallas import tpu as pltpu
from jax.experimental.pallas import tpu_sc as plsc

VSIZE = 1000000
ESIZE = 16
PK = 2 * ESIZE
SENTLEN = 50
RAD = 5
NSAMPL = 5
BATCH = 4096

NHALF = 2
BATCH_H = BATCH // NHALF

NC, NS = 2, 16
NW = NC * NS
CHUNK = 128
NCHUNK = 64
WINDOW = 16

ROWS_W = BATCH_H * SENTLEN // NW
NCH = ROWS_W // CHUNK
NROWS_NEG_W = BATCH_H * NSAMPL // NW
NCH_NEG = NROWS_NEG_W // NCHUNK


def _sc_gather_body(tab_hbm, sent_hbm, negw_hbm, cc_out, neg_out,
                    idx_v, nidx_v, rows_v, sem):
    wid = lax.axis_index("s") * NC + lax.axis_index("c")
    pltpu.sync_copy(sent_hbm.at[wid], idx_v)
    pltpu.sync_copy(negw_hbm.at[wid], nidx_v)

    def gather_to(out_hbm, idx_ref, nch, chunk):
        def body(j, carry):
            pltpu.async_copy(tab_hbm.at[idx_ref.at[j]],
                             rows_v.at[pl.ds(j * chunk, chunk)], sem)

            @pl.when(j >= WINDOW)
            def _():
                pltpu.make_async_copy(tab_hbm.at[pl.ds(0, chunk)],
                                      rows_v.at[pl.ds(0, chunk)], sem).wait()

            return carry

        lax.fori_loop(0, nch, body, 0)
        tail = min(nch, WINDOW) * chunk
        pltpu.make_async_copy(tab_hbm.at[pl.ds(0, tail)],
                              rows_v.at[pl.ds(0, tail)], sem).wait()
        pltpu.sync_copy(rows_v.at[pl.ds(0, nch * chunk)], out_hbm.at[wid])

    gather_to(cc_out, idx_v, NCH, CHUNK)
    gather_to(neg_out, nidx_v, NCH_NEG, NCHUNK)


@functools.cache
def _make_sc_gather():
    return pl.kernel(
        _sc_gather_body,
        out_type=[
            jax.ShapeDtypeStruct((NW, ROWS_W, PK), jnp.bfloat16),
            jax.ShapeDtypeStruct((NW, NROWS_NEG_W, PK), jnp.bfloat16),
        ],
        mesh=plsc.VectorSubcoreMesh(core_axis_name="c", subcore_axis_name="s",
                                    num_cores=NC, num_subcores=NS),
        scratch_types=[
            pltpu.VMEM((NCH, CHUNK), jnp.int32),
            pltpu.VMEM((NCH_NEG, NCHUNK), jnp.int32),
            pltpu.VMEM((ROWS_W, PK), jnp.bfloat16),
            pltpu.SemaphoreType.DMA,
        ],
        compiler_params=pltpu.CompilerParams(use_tc_tiling_on_sc=False),
    )


BB = 512
LP = SENTLEN * PK
NP = NSAMPL * PK


def _softplus(x):
    return jnp.log1p(jnp.exp(-jnp.abs(x))) + jnp.maximum(x, 0.0)


def _tc_loss_body(cc_ref, neg_ref, out_ref):
    i = pl.program_id(0)
    x = cc_ref[...].astype(jnp.float32)
    y = neg_ref[...].astype(jnp.float32)
    ki = lax.broadcasted_iota(jnp.int32, (LP, SENTLEN), 0)
    ji = lax.broadcasted_iota(jnp.int32, (LP, SENTLEN), 1)
    Scen = ((ki // PK == ji) & (ki % PK < ESIZE)).astype(jnp.float32)
    Scon = ((ki // PK == ji) & (ki % PK >= ESIZE)).astype(jnp.float32)
    pos = jnp.zeros((), jnp.float32)
    for d in range(1, RAD + 1):
        w1 = LP - PK * d - ESIZE
        p1 = x[:, :w1] * x[:, PK * d + ESIZE:]
        s1 = jnp.dot(p1, Scen[:w1, :SENTLEN - d],
                     preferred_element_type=jnp.float32)
        w2 = LP - PK * d + ESIZE
        p2 = x[:, :w2] * x[:, PK * d - ESIZE:]
        s2 = jnp.dot(p2, Scon[:w2, :SENTLEN - d],
                     preferred_element_type=jnp.float32)
        pos += jnp.sum(_softplus(-s1)) + jnp.sum(_softplus(-s2))
    negsum = jnp.zeros((), jnp.float32)
    kn = lax.broadcasted_iota(jnp.int32, (NP, LP), 0)
    mn = lax.broadcasted_iota(jnp.int32, (NP, LP), 1)
    for n in range(NSAMPL):
        Tn = ((kn == PK * n + ESIZE + mn % PK) & (mn % PK < ESIZE)
              ).astype(jnp.float32)
        nb = jnp.dot(y, Tn, preferred_element_type=jnp.float32)
        s = jnp.dot(x * nb, Scen, preferred_element_type=jnp.float32)
        negsum += jnp.sum(_softplus(s))
    val = (pos / (BATCH * SENTLEN * SENTLEN)
           + negsum / (BATCH * SENTLEN * NSAMPL))

    @pl.when(i == 0)
    def _():
        out_ref[...] = jnp.zeros((1, 1), jnp.float32)

    out_ref[...] = out_ref[...] + val


_tc_loss = pl.pallas_call(
    _tc_loss_body,
    grid=(BATCH_H // BB,),
    in_specs=[
        pl.BlockSpec((BB, LP), lambda i: (i, 0)),
        pl.BlockSpec((BB, NP), lambda i: (i, 0)),
    ],
    out_specs=pl.BlockSpec((1, 1), lambda i: (0, 0)),
    out_shape=jax.ShapeDtypeStruct((1, 1), jnp.float32),
)


def kernel(sent, cenb_w, cemb_w, negwords):
    tab = jnp.concatenate([cenb_w.astype(jnp.bfloat16),
                           cemb_w.astype(jnp.bfloat16)], axis=1)
    sent_r = sent.astype(jnp.int32).reshape(NHALF, NW, NCH, CHUNK)
    negw_r = negwords.astype(jnp.int32).reshape(NHALF, NW, NCH_NEG, NCHUNK)
    sc = _make_sc_gather()
    total = jnp.zeros((), jnp.float32)
    for h in range(NHALF):
        cc_g, neg_g = sc(tab, sent_r[h], negw_r[h])
        cc = cc_g.reshape(BATCH_H, LP)
        neg = neg_g.reshape(BATCH_H, NP)
        total = total + _tc_loss(cc, neg)[0, 0]
    return total
